# Initial kernel scaffold; baseline (speedup 1.0000x reference)
import jax
import jax.numpy as jnp
from jax.experimental import pallas as pl


def kernel(x, edge_index, edge_attr, ln_gamma, ln_beta, W1, b1, bn_gamma, bn_beta, W2, b2, bond_emb):
    raise NotImplementedError("write your pallas kernel here")


# speedup vs baseline: 2.9560x; 2.9560x over previous
import functools

import jax
import jax.numpy as jnp
from jax import lax
from jax.experimental import pallas as pl
from jax.experimental.pallas import tpu as pltpu
from jax.experimental.pallas import tpu_sc as plsc

N = 10000
E = 320000
D = 128
NC = 2
NS = 16
NW = NC * NS
CH = 128
EPT = 10240
NCHUNK = EPT // CH
EPAD = NW * EPT
NPAD = 10240
ROWS_INIT = NPAD // NS
EPS = 1e-5


def _sc_agg_body(hn, tc, srcs, aidx, dsts, zeros, out,
                 agg_sh, src_v, a_v, dst_v, msg_v, sem):
    cid = lax.axis_index("c")
    sid = lax.axis_index("s")
    wid = cid * NS + sid

    pltpu.sync_copy(zeros, msg_v)

    def zinit(t, carry):
        pltpu.sync_copy(msg_v, agg_sh.at[pl.ds(sid * ROWS_INIT + t * CH, CH)])
        return carry

    lax.fori_loop(0, ROWS_INIT // CH, zinit, 0)

    pltpu.sync_copy(srcs.at[wid], src_v)
    pltpu.sync_copy(aidx.at[wid], a_v)
    pltpu.sync_copy(dsts.at[wid], dst_v)
    plsc.subcore_barrier()

    def chunk(j, carry):
        pltpu.async_copy(tc.at[a_v.at[j]], msg_v, sem).wait()
        pltpu.async_copy(hn.at[src_v.at[j]], msg_v, sem, add=True).wait()

        def relu_row(r, c2):
            for k in range(D // 16):
                s = (r, pl.ds(k * 16, 16))
                msg_v[s] = jnp.maximum(msg_v[s], 0.0)
            return c2

        lax.fori_loop(0, CH, relu_row, 0)
        pltpu.sync_copy(msg_v, agg_sh.at[dst_v.at[j]], add=True)
        return carry

    lax.fori_loop(0, NCHUNK, chunk, 0)
    plsc.subcore_barrier()

    def cout(t, carry):
        base = sid * ROWS_INIT + t * CH
        pltpu.sync_copy(agg_sh.at[pl.ds(base, CH)], msg_v)
        pltpu.sync_copy(msg_v, out.at[pl.ds(cid * NPAD + base, CH)])
        return carry

    lax.fori_loop(0, ROWS_INIT // CH, cout, 0)


_sc_agg = functools.partial(
    pl.kernel,
    out_type=jax.ShapeDtypeStruct((NC * NPAD, D), jnp.float32),
    mesh=plsc.VectorSubcoreMesh(core_axis_name="c", subcore_axis_name="s",
                                num_cores=NC, num_subcores=NS),
    scratch_types=[
        pltpu.VMEM_SHARED((NPAD, D), jnp.float32),
        pltpu.VMEM((NCHUNK, CH), jnp.int32),
        pltpu.VMEM((NCHUNK, CH), jnp.int32),
        pltpu.VMEM((NCHUNK, CH), jnp.int32),
        pltpu.VMEM((CH, D), jnp.float32),
        pltpu.SemaphoreType.DMA,
    ],
)(_sc_agg_body)


def _bn_body(h_ref, g_ref, b_ref, o_ref):
    h = h_ref[...]
    m = jnp.mean(h, axis=0, keepdims=True)
    v = jnp.mean((h - m) ** 2, axis=0, keepdims=True)
    o_ref[...] = (h - m) * lax.rsqrt(v + EPS) * g_ref[...] + b_ref[...]


_bn = pl.pallas_call(
    _bn_body, out_shape=jax.ShapeDtypeStruct((N, D), jnp.float32))


def _mlp_body(hn_ref, p_ref, w1_ref, b1_ref, bng_ref, bnb_ref,
              w2_ref, b2_ref, lng_ref, lnb_ref, o_ref, *, last):
    t = hn_ref[...] + p_ref[0, :N] + p_ref[1, :N]
    y = jnp.dot(t, w1_ref[...], preferred_element_type=jnp.float32) + b1_ref[...]
    m = jnp.mean(y, axis=0, keepdims=True)
    v = jnp.mean((y - m) ** 2, axis=0, keepdims=True)
    y = (y - m) * lax.rsqrt(v + EPS) * bng_ref[...] + bnb_ref[...]
    y = jnp.maximum(y, 0.0)
    y = jnp.dot(y, w2_ref[...], preferred_element_type=jnp.float32) + b2_ref[...]
    if last:
        o_ref[...] = y
    else:
        h = jnp.maximum(y, 0.0)
        m2 = jnp.mean(h, axis=0, keepdims=True)
        v2 = jnp.mean((h - m2) ** 2, axis=0, keepdims=True)
        o_ref[...] = (h - m2) * lax.rsqrt(v2 + EPS) * lng_ref[...] + lnb_ref[...]


def _make_mlp(last):
    return pl.pallas_call(
        functools.partial(_mlp_body, last=last),
        out_shape=jax.ShapeDtypeStruct((N, D), jnp.float32))


_mlp_mid = _make_mlp(False)
_mlp_last = _make_mlp(True)


def kernel(x, edge_index, edge_attr, ln_gamma, ln_beta, W1, b1,
           bn_gamma, bn_beta, W2, b2, bond_emb):
    L = W1.shape[0]
    src = edge_index[0]
    dst = edge_index[1]
    a = edge_attr[:, 0] * 64 + edge_attr[:, 1] * 8 + edge_attr[:, 2]
    pad = EPAD - E
    srcp = jnp.concatenate([src, jnp.zeros((pad,), jnp.int32)]).reshape(NW, NCHUNK, CH)
    ap = jnp.concatenate([a, jnp.zeros((pad,), jnp.int32)]).reshape(NW, NCHUNK, CH)
    dstp = jnp.concatenate([dst, jnp.full((pad,), N, jnp.int32)]).reshape(NW, NCHUNK, CH)
    tc_all = (bond_emb[:, 0, :, None, None, :]
              + bond_emb[:, 1, None, :, None, :]
              + bond_emb[:, 2, None, None, :, :]).reshape(L, 512, D)
    zeros = jnp.zeros((CH, D), jnp.float32)
    g2 = ln_gamma.reshape(L, 1, D)
    b2_ = ln_beta.reshape(L, 1, D)
    bng2 = bn_gamma.reshape(L, 1, D)
    bnb2 = bn_beta.reshape(L, 1, D)
    b1r = b1.reshape(L, 1, D)
    b2r = b2.reshape(L, 1, D)

    hn = _bn(x, g2[0], b2_[0])
    for l in range(L):
        parts = _sc_agg(hn, tc_all[l], srcp, ap, dstp, zeros).reshape(2, NPAD, D)
        w1t = W1[l].T
        w2t = W2[l].T
        if l == L - 1:
            hn = _mlp_last(hn, parts, w1t, b1r[l], bng2[l], bnb2[l],
                           w2t, b2r[l], g2[l], b2_[l])
        else:
            hn = _mlp_mid(hn, parts, w1t, b1r[l], bng2[l], bnb2[l],
                          w2t, b2r[l], g2[l + 1], b2_[l + 1])
    return hn
